# baseline (speedup 1.0000x reference)
import functools

import jax
import jax.numpy as jnp
from jax.experimental import pallas as pl

_KT = 2048


def _mm_kernel(x_ref, w_ref, b_ref, o_ref, *, nsteps, k_total):
    k = pl.program_id(0)

    @pl.when(k == 0)
    def _():
        o_ref[...] = jnp.broadcast_to(b_ref[...], o_ref.shape)

    @pl.when(k < nsteps - 1)
    def _():
        o_ref[...] += jnp.dot(
            x_ref[...], w_ref[...], preferred_element_type=jnp.float32
        )

    @pl.when(k == nsteps - 1)
    def _():
        valid = k_total - k * _KT
        x = x_ref[...]
        w = w_ref[...]
        xcol = jax.lax.broadcasted_iota(jnp.int32, x.shape, 1)
        wrow = jax.lax.broadcasted_iota(jnp.int32, w.shape, 0)
        x = jnp.where(xcol < valid, x, 0.0)
        w = jnp.where(wrow < valid, w, 0.0)
        o_ref[...] += jnp.dot(x, w, preferred_element_type=jnp.float32)


def kernel(X, W, b):
    M, K = X.shape
    N = W.shape[1]
    nsteps = pl.cdiv(K, _KT)
    b2 = b.reshape(1, N)
    return pl.pallas_call(
        functools.partial(_mm_kernel, nsteps=nsteps, k_total=K),
        grid=(nsteps,),
        in_specs=[
            pl.BlockSpec((M, _KT), lambda k: (0, k)),
            pl.BlockSpec((_KT, N), lambda k: (k, 0)),
            pl.BlockSpec((1, N), lambda k: (0, 0)),
        ],
        out_specs=pl.BlockSpec((M, N), lambda k: (0, 0)),
        out_shape=jax.ShapeDtypeStruct((M, N), jnp.float32),
    )(X, W, b2)
